# baseline (speedup 1.0000x reference)
import functools

import jax
import jax.numpy as jnp
from jax import lax
from jax.experimental import pallas as pl
from jax.experimental.pallas import tpu as pltpu
from jax.experimental.pallas import tpu_sc as plsc

N = 10000
D = 128
E = 320000
NPAD = 10240
NC = 2
NS = 16
EPT0 = E // NS
B = 32
U = 4 * B
MAXU = -(-EPT0 // U) + 1
CAPC = 4 * MAXU
HALF = 5000
TROWS = HALF + 8
RPT = 624
TAIL = N - NS * RPT
TLR = 312


def _tc_transform(x_user_p, x_item_p, W_msg_u2i, W_self_u2i, W_msg_i2u, W_self_i2u):
    BLK = 256

    def body(xu_ref, xi_ref, wm0_ref, ws0_ref, wm1_ref, ws1_ref, y_ref, b_ref):
        xu = xu_ref[...]
        xi = xi_ref[...]
        y_ref[0] = jnp.dot(xu, wm0_ref[...], preferred_element_type=jnp.float32)
        y_ref[1] = jnp.dot(xi, wm1_ref[...], preferred_element_type=jnp.float32)
        b_ref[0] = jnp.dot(xi, ws0_ref[...], preferred_element_type=jnp.float32)
        b_ref[1] = jnp.dot(xu, ws1_ref[...], preferred_element_type=jnp.float32)

    grid = (NPAD // BLK,)
    w_spec = pl.BlockSpec((D, D), lambda i: (0, 0))
    return pl.pallas_call(
        body,
        grid=grid,
        in_specs=[
            pl.BlockSpec((BLK, D), lambda i: (i, 0)),
            pl.BlockSpec((BLK, D), lambda i: (i, 0)),
            w_spec, w_spec, w_spec, w_spec,
        ],
        out_specs=[
            pl.BlockSpec((NC, BLK, D), lambda i: (0, i, 0)),
            pl.BlockSpec((NC, BLK, D), lambda i: (0, i, 0)),
        ],
        out_shape=[
            jax.ShapeDtypeStruct((NC, NPAD, D), jnp.float32),
            jax.ShapeDtypeStruct((NC, NPAD, D), jnp.float32),
        ],
    )(x_user_p, x_item_p, W_msg_u2i, W_self_u2i, W_msg_i2u, W_self_i2u)


def _sc_conv(y_all, e_src, e_dst, counts, base_all):
    mesh = plsc.VectorSubcoreMesh(core_axis_name="c", subcore_axis_name="s")

    @functools.partial(
        pl.kernel,
        out_type=(
            jax.ShapeDtypeStruct((N, D), jnp.float32),
            jax.ShapeDtypeStruct((N, D), jnp.float32),
        ),
        mesh=mesh,
        scratch_types=[
            pltpu.VMEM((1, B), jnp.int32),
            pltpu.VMEM((1, B), jnp.int32),
            pltpu.VMEM((1, B), jnp.int32),
            pltpu.VMEM((1, B), jnp.int32),
            pltpu.VMEM((1, B), jnp.int32),
            pltpu.VMEM((1, B), jnp.int32),
            pltpu.VMEM((1, B), jnp.int32),
            pltpu.VMEM((1, B), jnp.int32),
            pltpu.VMEM((B, D), jnp.float32),
            pltpu.VMEM((B, D), jnp.float32),
            pltpu.VMEM((8, 16), jnp.int32),
            pltpu.SemaphoreType.DMA,
            pltpu.SemaphoreType.DMA,
            pltpu.SemaphoreType.DMA,
            pltpu.SemaphoreType.DMA,
            pltpu.SemaphoreType.DMA,
            pltpu.SemaphoreType.DMA,
            pltpu.VMEM_SHARED((TROWS, D), jnp.float32),
            pltpu.VMEM_SHARED((N, D), jnp.float32),
        ],
    )
    def k(y_ref, es_ref, ed_ref, cnt_ref, base_ref, out_user, out_item,
          si0, si1, si2, si3, di0, di1, di2, di3, r0, r1, cntv,
          g0, g1, i0, i1, i2, i3, tbl, acc):
        cid = lax.axis_index("c")
        sid = lax.axis_index("s")
        SI = (si0, si1, si2, si3)
        DI = (di0, di1, di2, di3)
        RW = (r0, r1)
        GS = (g0, g1)
        IS = (i0, i1, i2, i3)

        row0 = pl.multiple_of(sid * RPT, 8)
        pltpu.sync_copy(base_ref.at[cid, pl.ds(row0, RPT)], acc.at[pl.ds(row0, RPT)])

        @pl.when(sid == NS - 1)
        def _():
            pltpu.sync_copy(base_ref.at[cid, pl.ds(NS * RPT, TAIL)],
                            acc.at[pl.ds(NS * RPT, TAIL)])

        pltpu.sync_copy(cnt_ref.at[cid, sid], cntv)
        cv = cntv[0]
        n4_0 = cv[0]
        start1 = cv[1]
        n4_1 = cv[2]

        @pl.when(sid == 0)
        def _():
            pltpu.sync_copy(y_ref.at[cid, pl.ds(N, 8)], tbl.at[pl.ds(HALF, 8)])

        def load_table(p):
            off = pl.multiple_of(p * HALF + sid * TLR, 8)
            dst_off = pl.multiple_of(sid * TLR, 8)
            pltpu.sync_copy(y_ref.at[cid, pl.ds(off, TLR)],
                            tbl.at[pl.ds(dst_off, TLR)])

            @pl.when(sid == NS - 1)
            def _():
                o2 = pl.multiple_of(p * HALF + NS * TLR, 8)
                rem = HALF - NS * TLR
                pltpu.sync_copy(y_ref.at[cid, pl.ds(o2, rem)],
                                tbl.at[pl.ds(NS * TLR, rem)])

        def phase(s0, n4):
            nch = n4 * 4

            def iload(c, slot):
                pltpu.async_copy(es_ref.at[cid, sid, c], SI[slot], IS[slot])
                pltpu.async_copy(ed_ref.at[cid, sid, c], DI[slot], IS[slot])

            def iwait(slot):
                pltpu.make_async_copy(es_ref.at[cid, sid, 0], SI[slot], IS[slot]).wait()
                pltpu.make_async_copy(ed_ref.at[cid, sid, 0], DI[slot], IS[slot]).wait()

            def gath(slot, b):
                pltpu.async_copy(tbl.at[SI[slot].at[0]], RW[b], GS[b])

            def gwait(slot, b):
                pltpu.make_async_copy(tbl.at[SI[slot].at[0]], RW[b], GS[b]).wait()

            @pl.when(n4 > 0)
            def _():
                pltpu.sync_copy(es_ref.at[cid, sid, s0], SI[0])
                pltpu.sync_copy(ed_ref.at[cid, sid, s0], DI[0])
                pltpu.sync_copy(es_ref.at[cid, sid, s0 + 1], SI[1])
                pltpu.sync_copy(ed_ref.at[cid, sid, s0 + 1], DI[1])
                iload(s0 + 2, 2)
                iload(s0 + 3, 3)
                gath(0, 0)
                gath(1, 1)

            def body(i, carry):
                jb = i * 4
                for sub in range(4):
                    j = jb + sub
                    b = sub % 2
                    gwait(sub, b)
                    pltpu.sync_copy(RW[b], acc.at[DI[sub].at[0]], add=True)

                    @pl.when(j + 4 < nch)
                    def _():
                        iload(s0 + j + 4, sub)

                    @pl.when(j + 2 < nch)
                    def _():
                        iwait((sub + 2) % 4)
                        gath((sub + 2) % 4, b)
                return carry

            lax.fori_loop(0, n4, body, 0)

        load_table(0)
        plsc.subcore_barrier()
        phase(0, n4_0)
        plsc.subcore_barrier()
        load_table(1)
        plsc.subcore_barrier()
        phase(start1, n4_1)
        plsc.subcore_barrier()

        @pl.when(cid == 0)
        def _():
            pltpu.sync_copy(acc.at[pl.ds(row0, RPT)], out_item.at[pl.ds(row0, RPT)])

            @pl.when(sid == NS - 1)
            def _():
                pltpu.sync_copy(acc.at[pl.ds(NS * RPT, TAIL)],
                                out_item.at[pl.ds(NS * RPT, TAIL)])

        @pl.when(cid == 1)
        def _():
            pltpu.sync_copy(acc.at[pl.ds(row0, RPT)], out_user.at[pl.ds(row0, RPT)])

            @pl.when(sid == NS - 1)
            def _():
                pltpu.sync_copy(acc.at[pl.ds(NS * RPT, TAIL)],
                                out_user.at[pl.ds(NS * RPT, TAIL)])

    return k(y_all, e_src, e_dst, counts, base_all)


def _prep_edges(edge_index_u2i, edge_index_i2u):
    src = jnp.stack([edge_index_u2i[0].astype(jnp.int32),
                     edge_index_i2u[0].astype(jnp.int32)]).reshape(NC, NS, EPT0)
    dst = jnp.stack([edge_index_u2i[1].astype(jnp.int32),
                     edge_index_i2u[1].astype(jnp.int32)]).reshape(NC, NS, EPT0)
    hi = src >= HALF
    clo = jnp.cumsum(~hi, axis=-1, dtype=jnp.int32)
    chi = jnp.cumsum(hi, axis=-1, dtype=jnp.int32)
    nlo = clo[..., -1]
    n4_0 = -(-nlo // U)
    start1e = n4_0 * U
    pos = jnp.where(hi, start1e[..., None] + chi - 1, clo - 1)
    cap_e = CAPC * B
    tile_base = (jnp.arange(NC, dtype=jnp.int32)[:, None, None] * NS
                 + jnp.arange(NS, dtype=jnp.int32)[None, :, None]) * cap_e
    flat_pos = (tile_base + pos).ravel()
    src_v = jnp.where(hi, src - HALF, src).ravel()
    out_src = jnp.full((NC * NS * cap_e,), HALF, jnp.int32).at[flat_pos].set(
        src_v, unique_indices=True)
    out_dst = jnp.zeros((NC * NS * cap_e,), jnp.int32).at[flat_pos].set(
        dst.ravel(), unique_indices=True)
    n4_1 = -(-(EPT0 - nlo) // U)
    counts = jnp.zeros((NC, NS, 8, 16), jnp.int32)
    counts = counts.at[:, :, 0, 0].set(n4_0)
    counts = counts.at[:, :, 0, 1].set(n4_0 * 4)
    counts = counts.at[:, :, 0, 2].set(n4_1)
    return (out_src.reshape(NC, NS, CAPC, 1, B),
            out_dst.reshape(NC, NS, CAPC, 1, B),
            counts)


def kernel(x_user, x_item, edge_index_u2i, edge_index_i2u,
           W_msg_u2i, W_self_u2i, W_msg_i2u, W_self_i2u):
    x_user_p = jnp.pad(x_user, ((0, NPAD - N), (0, 0)))
    x_item_p = jnp.pad(x_item, ((0, NPAD - N), (0, 0)))
    e_src, e_dst, counts = _prep_edges(edge_index_u2i, edge_index_i2u)
    y_all, base_all = _tc_transform(x_user_p, x_item_p,
                                    W_msg_u2i, W_self_u2i, W_msg_i2u, W_self_i2u)
    out_user, out_item = _sc_conv(y_all, e_src, e_dst, counts, base_all)
    return (out_user, out_item)


# speedup vs baseline: 4.7729x; 4.7729x over previous
import functools

import jax
import jax.numpy as jnp
from jax import lax
from jax.experimental import pallas as pl
from jax.experimental.pallas import tpu as pltpu
from jax.experimental.pallas import tpu_sc as plsc

N = 10000
D = 128
E = 320000
NPAD = 10240
NC = 2
NS = 16
EPT0 = E // NS
B = 32
U = 4 * B
NU = -(-EPT0 // U)
CAPC = 4 * NU
HALF = 5000
TROWS = HALF + 8
RPT = 624
TAIL = N - NS * RPT
TLR = 312


def _tc_transform(x_user_p, x_item_p, W_msg_u2i, W_self_u2i, W_msg_i2u, W_self_i2u):
    BLK = 256

    def body(xu_ref, xi_ref, wm0_ref, ws0_ref, wm1_ref, ws1_ref, y_ref, b_ref):
        xu = xu_ref[...]
        xi = xi_ref[...]
        y_ref[0] = jnp.dot(xu, wm0_ref[...], preferred_element_type=jnp.float32)
        y_ref[1] = jnp.dot(xi, wm1_ref[...], preferred_element_type=jnp.float32)
        b_ref[0] = jnp.dot(xi, ws0_ref[...], preferred_element_type=jnp.float32)
        b_ref[1] = jnp.dot(xu, ws1_ref[...], preferred_element_type=jnp.float32)

    grid = (NPAD // BLK,)
    w_spec = pl.BlockSpec((D, D), lambda i: (0, 0))
    return pl.pallas_call(
        body,
        grid=grid,
        in_specs=[
            pl.BlockSpec((BLK, D), lambda i: (i, 0)),
            pl.BlockSpec((BLK, D), lambda i: (i, 0)),
            w_spec, w_spec, w_spec, w_spec,
        ],
        out_specs=[
            pl.BlockSpec((NC, BLK, D), lambda i: (0, i, 0)),
            pl.BlockSpec((NC, BLK, D), lambda i: (0, i, 0)),
        ],
        out_shape=[
            jax.ShapeDtypeStruct((NC, NPAD, D), jnp.float32),
            jax.ShapeDtypeStruct((NC, NPAD, D), jnp.float32),
        ],
    )(x_user_p, x_item_p, W_msg_u2i, W_self_u2i, W_msg_i2u, W_self_i2u)


def _sc_conv(y_all, e_src, e_dst, counts, base_all):
    mesh = plsc.VectorSubcoreMesh(core_axis_name="c", subcore_axis_name="s")

    @functools.partial(
        pl.kernel,
        out_type=(
            jax.ShapeDtypeStruct((N, D), jnp.float32),
            jax.ShapeDtypeStruct((N, D), jnp.float32),
        ),
        mesh=mesh,
        scratch_types=[
            pltpu.VMEM((1, B), jnp.int32),
            pltpu.VMEM((1, B), jnp.int32),
            pltpu.VMEM((1, B), jnp.int32),
            pltpu.VMEM((1, B), jnp.int32),
            pltpu.VMEM((1, B), jnp.int32),
            pltpu.VMEM((1, B), jnp.int32),
            pltpu.VMEM((1, B), jnp.int32),
            pltpu.VMEM((1, B), jnp.int32),
            pltpu.VMEM((B, D), jnp.float32),
            pltpu.VMEM((B, D), jnp.float32),
            pltpu.VMEM((8, 16), jnp.int32),
            pltpu.SemaphoreType.DMA,
            pltpu.SemaphoreType.DMA,
            pltpu.SemaphoreType.DMA,
            pltpu.SemaphoreType.DMA,
            pltpu.SemaphoreType.DMA,
            pltpu.SemaphoreType.DMA,
            pltpu.VMEM_SHARED((TROWS, D), jnp.float32),
            pltpu.VMEM_SHARED((N, D), jnp.float32),
        ],
    )
    def k(y_ref, es0_ref, es1_ref, ed_ref, cnt_ref, base_ref, out_user, out_item,
          si0, si1, si2, si3, di0, di1, di2, di3, r0, r1, cntv,
          g0, g1, i0, i1, i2, i3, tbl, acc):
        cid = lax.axis_index("c")
        sid = lax.axis_index("s")
        SI = (si0, si1, si2, si3)
        DI = (di0, di1, di2, di3)
        RW = (r0, r1)
        GS = (g0, g1)
        IS = (i0, i1, i2, i3)

        row0 = pl.multiple_of(sid * RPT, 8)
        pltpu.sync_copy(base_ref.at[cid, pl.ds(row0, RPT)], acc.at[pl.ds(row0, RPT)])

        @pl.when(sid == NS - 1)
        def _():
            pltpu.sync_copy(base_ref.at[cid, pl.ds(NS * RPT, TAIL)],
                            acc.at[pl.ds(NS * RPT, TAIL)])

        pltpu.sync_copy(cnt_ref.at[cid, sid], cntv)
        cv = cntv[0]
        n4_0 = cv[0]
        start1 = cv[1]
        n4_1 = cv[2]

        @pl.when(sid == 0)
        def _():
            pltpu.sync_copy(y_ref.at[cid, pl.ds(N, 8)], tbl.at[pl.ds(HALF, 8)])

        def load_table(p):
            off = pl.multiple_of(p * HALF + sid * TLR, 8)
            dst_off = pl.multiple_of(sid * TLR, 8)
            pltpu.sync_copy(y_ref.at[cid, pl.ds(off, TLR)],
                            tbl.at[pl.ds(dst_off, TLR)])

            @pl.when(sid == NS - 1)
            def _():
                o2 = pl.multiple_of(p * HALF + NS * TLR, 8)
                rem = HALF - NS * TLR
                pltpu.sync_copy(y_ref.at[cid, pl.ds(o2, rem)],
                                tbl.at[pl.ds(NS * TLR, rem)])

        def phase(es_ref, s0, n4):
            nch = n4 * 4

            def iload(c, slot):
                pltpu.async_copy(es_ref.at[cid, sid, c], SI[slot], IS[slot])
                pltpu.async_copy(ed_ref.at[cid, sid, c], DI[slot], IS[slot])

            def iwait(slot):
                pltpu.make_async_copy(es_ref.at[cid, sid, 0], SI[slot], IS[slot]).wait()
                pltpu.make_async_copy(ed_ref.at[cid, sid, 0], DI[slot], IS[slot]).wait()

            def gath(slot, b):
                pltpu.async_copy(tbl.at[SI[slot].at[0]], RW[b], GS[b])

            def gwait(slot, b):
                pltpu.make_async_copy(tbl.at[SI[slot].at[0]], RW[b], GS[b]).wait()

            @pl.when(n4 > 0)
            def _():
                pltpu.sync_copy(es_ref.at[cid, sid, s0], SI[0])
                pltpu.sync_copy(ed_ref.at[cid, sid, s0], DI[0])
                pltpu.sync_copy(es_ref.at[cid, sid, s0 + 1], SI[1])
                pltpu.sync_copy(ed_ref.at[cid, sid, s0 + 1], DI[1])
                iload(s0 + 2, 2)
                iload(s0 + 3, 3)
                gath(0, 0)
                gath(1, 1)

            def body(i, carry):
                jb = i * 4
                for sub in range(4):
                    j = jb + sub
                    b = sub % 2
                    gwait(sub, b)
                    pltpu.sync_copy(RW[b], acc.at[DI[sub].at[0]], add=True)

                    @pl.when(j + 4 < nch)
                    def _():
                        iload(s0 + j + 4, sub)

                    @pl.when(j + 2 < nch)
                    def _():
                        iwait((sub + 2) % 4)
                        gath((sub + 2) % 4, b)
                return carry

            lax.fori_loop(0, n4, body, 0)

        load_table(0)
        plsc.subcore_barrier()
        phase(es0_ref, 0, n4_0)
        plsc.subcore_barrier()
        load_table(1)
        plsc.subcore_barrier()
        phase(es1_ref, start1, n4_1)
        plsc.subcore_barrier()

        @pl.when(cid == 0)
        def _():
            pltpu.sync_copy(acc.at[pl.ds(row0, RPT)], out_item.at[pl.ds(row0, RPT)])

            @pl.when(sid == NS - 1)
            def _():
                pltpu.sync_copy(acc.at[pl.ds(NS * RPT, TAIL)],
                                out_item.at[pl.ds(NS * RPT, TAIL)])

        @pl.when(cid == 1)
        def _():
            pltpu.sync_copy(acc.at[pl.ds(row0, RPT)], out_user.at[pl.ds(row0, RPT)])

            @pl.when(sid == NS - 1)
            def _():
                pltpu.sync_copy(acc.at[pl.ds(NS * RPT, TAIL)],
                                out_user.at[pl.ds(NS * RPT, TAIL)])

    return k(y_all, e_src[0], e_src[1], e_dst, counts, base_all)


def _prep_edges(edge_index_u2i, edge_index_i2u):
    src = jnp.stack([edge_index_u2i[0].astype(jnp.int32),
                     edge_index_i2u[0].astype(jnp.int32)]).reshape(NC, NS, EPT0)
    dst = jnp.stack([edge_index_u2i[1].astype(jnp.int32),
                     edge_index_i2u[1].astype(jnp.int32)]).reshape(NC, NS, EPT0)
    key = (src >= HALF).astype(jnp.int32)
    s_key, s_src, s_dst = lax.sort((key, src, dst), dimension=-1,
                                   is_stable=True, num_keys=1)
    lo = s_key == 0
    es0 = jnp.where(lo, s_src, HALF)
    es1 = jnp.where(lo, HALF, s_src - HALF)
    cap_e = CAPC * B
    padn = cap_e - EPT0
    pad_cfg = ((0, 0), (0, 0), (0, padn))
    es0 = jnp.pad(es0, pad_cfg, constant_values=HALF)
    es1 = jnp.pad(es1, pad_cfg, constant_values=HALF)
    ed = jnp.pad(s_dst, pad_cfg)
    nlo = jnp.sum(lo, axis=-1, dtype=jnp.int32)
    u0 = -(-nlo // U)
    s1 = nlo // U
    counts = jnp.zeros((NC, NS, 8, 16), jnp.int32)
    counts = counts.at[:, :, 0, 0].set(u0)
    counts = counts.at[:, :, 0, 1].set(s1 * 4)
    counts = counts.at[:, :, 0, 2].set(NU - s1)
    e_src = jnp.stack([es0, es1]).reshape(2, NC, NS, CAPC, 1, B)
    return e_src, ed.reshape(NC, NS, CAPC, 1, B), counts


def kernel(x_user, x_item, edge_index_u2i, edge_index_i2u,
           W_msg_u2i, W_self_u2i, W_msg_i2u, W_self_i2u):
    x_user_p = jnp.pad(x_user, ((0, NPAD - N), (0, 0)))
    x_item_p = jnp.pad(x_item, ((0, NPAD - N), (0, 0)))
    e_src, e_dst, counts = _prep_edges(edge_index_u2i, edge_index_i2u)
    y_all, base_all = _tc_transform(x_user_p, x_item_p,
                                    W_msg_u2i, W_self_u2i, W_msg_i2u, W_self_i2u)
    out_user, out_item = _sc_conv(y_all, e_src, e_dst, counts, base_all)
    return (out_user, out_item)


# speedup vs baseline: 5.2450x; 1.0989x over previous
import functools

import jax
import jax.numpy as jnp
from jax import lax
from jax.experimental import pallas as pl
from jax.experimental.pallas import tpu as pltpu
from jax.experimental.pallas import tpu_sc as plsc

N = 10000
D = 128
E = 320000
NPAD = 10240
NC = 2
NS = 16
EPT0 = E // NS
B = 32
U = 4 * B
NU = -(-EPT0 // U)
CAPC = 4 * NU
HALF = 5000
TROWS = HALF + 8
RPT = 624
TAIL = N - NS * RPT
TLR = 312


def _tc_transform(x_user_p, x_item_p, W_msg_u2i, W_self_u2i, W_msg_i2u, W_self_i2u):
    BLK = 256

    def body(xu_ref, xi_ref, wm0_ref, ws0_ref, wm1_ref, ws1_ref, y_ref, b_ref):
        xu = xu_ref[...]
        xi = xi_ref[...]
        y_ref[0] = jnp.dot(xu, wm0_ref[...], preferred_element_type=jnp.float32)
        y_ref[1] = jnp.dot(xi, wm1_ref[...], preferred_element_type=jnp.float32)
        b_ref[0] = jnp.dot(xi, ws0_ref[...], preferred_element_type=jnp.float32)
        b_ref[1] = jnp.dot(xu, ws1_ref[...], preferred_element_type=jnp.float32)

    grid = (NPAD // BLK,)
    w_spec = pl.BlockSpec((D, D), lambda i: (0, 0))
    return pl.pallas_call(
        body,
        grid=grid,
        in_specs=[
            pl.BlockSpec((BLK, D), lambda i: (i, 0)),
            pl.BlockSpec((BLK, D), lambda i: (i, 0)),
            w_spec, w_spec, w_spec, w_spec,
        ],
        out_specs=[
            pl.BlockSpec((NC, BLK, D), lambda i: (0, i, 0)),
            pl.BlockSpec((NC, BLK, D), lambda i: (0, i, 0)),
        ],
        out_shape=[
            jax.ShapeDtypeStruct((NC, NPAD, D), jnp.float32),
            jax.ShapeDtypeStruct((NC, NPAD, D), jnp.float32),
        ],
    )(x_user_p, x_item_p, W_msg_u2i, W_self_u2i, W_msg_i2u, W_self_i2u)


def _sc_conv(y_all, e_src, e_dst, counts, base_all):
    mesh = plsc.VectorSubcoreMesh(core_axis_name="c", subcore_axis_name="s")

    @functools.partial(
        pl.kernel,
        out_type=(
            jax.ShapeDtypeStruct((N, D), jnp.float32),
            jax.ShapeDtypeStruct((N, D), jnp.float32),
        ),
        mesh=mesh,
        scratch_types=[
            pltpu.VMEM((1, B), jnp.int32),
            pltpu.VMEM((1, B), jnp.int32),
            pltpu.VMEM((1, B), jnp.int32),
            pltpu.VMEM((1, B), jnp.int32),
            pltpu.VMEM((1, B), jnp.int32),
            pltpu.VMEM((1, B), jnp.int32),
            pltpu.VMEM((1, B), jnp.int32),
            pltpu.VMEM((1, B), jnp.int32),
            pltpu.VMEM((B, D), jnp.float32),
            pltpu.VMEM((B, D), jnp.float32),
            pltpu.VMEM((8, 16), jnp.int32),
            pltpu.SemaphoreType.DMA,
            pltpu.SemaphoreType.DMA,
            pltpu.SemaphoreType.DMA,
            pltpu.SemaphoreType.DMA,
            pltpu.SemaphoreType.DMA,
            pltpu.SemaphoreType.DMA,
            pltpu.VMEM_SHARED((TROWS, D), jnp.float32),
            pltpu.VMEM_SHARED((N, D), jnp.float32),
        ],
    )
    def k(y_ref, es0_ref, es1_ref, ed_ref, cnt_ref, base_ref, out_user, out_item,
          si0, si1, si2, si3, di0, di1, di2, di3, r0, r1, cntv,
          g0, g1, i0, i1, i2, i3, tbl, acc):
        cid = lax.axis_index("c")
        sid = lax.axis_index("s")
        SI = (si0, si1, si2, si3)
        DI = (di0, di1, di2, di3)
        RW = (r0, r1)
        GS = (g0, g1)
        IS = (i0, i1, i2, i3)

        row0 = pl.multiple_of(sid * RPT, 8)
        pltpu.sync_copy(base_ref.at[cid, pl.ds(row0, RPT)], acc.at[pl.ds(row0, RPT)])

        @pl.when(sid == NS - 1)
        def _():
            pltpu.sync_copy(base_ref.at[cid, pl.ds(NS * RPT, TAIL)],
                            acc.at[pl.ds(NS * RPT, TAIL)])

        pltpu.sync_copy(cnt_ref.at[cid, sid], cntv)
        cv = cntv[0]
        n4_0 = cv[0]
        start1 = cv[1]
        n4_1 = cv[2]

        @pl.when(sid == 0)
        def _():
            pltpu.sync_copy(y_ref.at[cid, pl.ds(N, 8)], tbl.at[pl.ds(HALF, 8)])

        def load_table(p):
            off = pl.multiple_of(p * HALF + sid * TLR, 8)
            dst_off = pl.multiple_of(sid * TLR, 8)
            pltpu.sync_copy(y_ref.at[cid, pl.ds(off, TLR)],
                            tbl.at[pl.ds(dst_off, TLR)])

            @pl.when(sid == NS - 1)
            def _():
                o2 = pl.multiple_of(p * HALF + NS * TLR, 8)
                rem = HALF - NS * TLR
                pltpu.sync_copy(y_ref.at[cid, pl.ds(o2, rem)],
                                tbl.at[pl.ds(NS * TLR, rem)])

        def phase(es_ref, s0, n4):
            nch = n4 * 4

            def iload(c, slot):
                pltpu.async_copy(es_ref.at[cid, sid, c], SI[slot], IS[slot])
                pltpu.async_copy(ed_ref.at[cid, sid, c], DI[slot], IS[slot])

            def iwait(slot):
                pltpu.make_async_copy(es_ref.at[cid, sid, 0], SI[slot], IS[slot]).wait()
                pltpu.make_async_copy(ed_ref.at[cid, sid, 0], DI[slot], IS[slot]).wait()

            def gath(slot, b):
                pltpu.async_copy(tbl.at[SI[slot].at[0]], RW[b], GS[b])

            def gwait(slot, b):
                pltpu.make_async_copy(tbl.at[SI[slot].at[0]], RW[b], GS[b]).wait()

            @pl.when(n4 > 0)
            def _():
                pltpu.sync_copy(es_ref.at[cid, sid, s0], SI[0])
                pltpu.sync_copy(ed_ref.at[cid, sid, s0], DI[0])
                pltpu.sync_copy(es_ref.at[cid, sid, s0 + 1], SI[1])
                pltpu.sync_copy(ed_ref.at[cid, sid, s0 + 1], DI[1])
                iload(s0 + 2, 2)
                iload(s0 + 3, 3)
                gath(0, 0)
                gath(1, 1)

            def body(i, carry):
                jb = i * 4
                for sub in range(4):
                    j = jb + sub
                    b = sub % 2
                    gwait(sub, b)
                    pltpu.sync_copy(RW[b], acc.at[DI[sub].at[0]], add=True)

                    @pl.when(j + 4 < nch)
                    def _():
                        iload(s0 + j + 4, sub)

                    @pl.when(j + 2 < nch)
                    def _():
                        iwait((sub + 2) % 4)
                        gath((sub + 2) % 4, b)
                return carry

            lax.fori_loop(0, n4, body, 0)

        load_table(0)
        plsc.subcore_barrier()
        phase(es0_ref, 0, n4_0)
        plsc.subcore_barrier()
        load_table(1)
        plsc.subcore_barrier()
        phase(es1_ref, start1, n4_1)
        plsc.subcore_barrier()

        @pl.when(cid == 0)
        def _():
            pltpu.sync_copy(acc.at[pl.ds(row0, RPT)], out_item.at[pl.ds(row0, RPT)])

            @pl.when(sid == NS - 1)
            def _():
                pltpu.sync_copy(acc.at[pl.ds(NS * RPT, TAIL)],
                                out_item.at[pl.ds(NS * RPT, TAIL)])

        @pl.when(cid == 1)
        def _():
            pltpu.sync_copy(acc.at[pl.ds(row0, RPT)], out_user.at[pl.ds(row0, RPT)])

            @pl.when(sid == NS - 1)
            def _():
                pltpu.sync_copy(acc.at[pl.ds(NS * RPT, TAIL)],
                                out_user.at[pl.ds(NS * RPT, TAIL)])

    return k(y_all, e_src[0], e_src[1], e_dst, counts, base_all)


def _prep_edges(edge_index_u2i, edge_index_i2u):
    src = jnp.stack([edge_index_u2i[0].astype(jnp.int32),
                     edge_index_i2u[0].astype(jnp.int32)]).reshape(NC, NS, EPT0)
    dst = jnp.stack([edge_index_u2i[1].astype(jnp.int32),
                     edge_index_i2u[1].astype(jnp.int32)]).reshape(NC, NS, EPT0)
    key = (src >= HALF).astype(jnp.int32)
    packed = (key << 28) | (src << 14) | dst
    s = lax.sort(packed, dimension=-1)
    s_src = (s >> 14) & 0x3FFF
    s_dst = s & 0x3FFF
    lo = s < (1 << 28)
    es0 = jnp.where(lo, s_src, HALF)
    es1 = jnp.where(lo, HALF, s_src - HALF)
    cap_e = CAPC * B
    padn = cap_e - EPT0
    pad_cfg = ((0, 0), (0, 0), (0, padn))
    es0 = jnp.pad(es0, pad_cfg, constant_values=HALF)
    es1 = jnp.pad(es1, pad_cfg, constant_values=HALF)
    ed = jnp.pad(s_dst, pad_cfg)
    nlo = jnp.sum(lo, axis=-1, dtype=jnp.int32)
    u0 = -(-nlo // U)
    s1 = nlo // U
    counts = jnp.zeros((NC, NS, 8, 16), jnp.int32)
    counts = counts.at[:, :, 0, 0].set(u0)
    counts = counts.at[:, :, 0, 1].set(s1 * 4)
    counts = counts.at[:, :, 0, 2].set(NU - s1)
    e_src = jnp.stack([es0, es1]).reshape(2, NC, NS, CAPC, 1, B)
    return e_src, ed.reshape(NC, NS, CAPC, 1, B), counts


def kernel(x_user, x_item, edge_index_u2i, edge_index_i2u,
           W_msg_u2i, W_self_u2i, W_msg_i2u, W_self_i2u):
    x_user_p = jnp.pad(x_user, ((0, NPAD - N), (0, 0)))
    x_item_p = jnp.pad(x_item, ((0, NPAD - N), (0, 0)))
    e_src, e_dst, counts = _prep_edges(edge_index_u2i, edge_index_i2u)
    y_all, base_all = _tc_transform(x_user_p, x_item_p,
                                    W_msg_u2i, W_self_u2i, W_msg_i2u, W_self_i2u)
    out_user, out_item = _sc_conv(y_all, e_src, e_dst, counts, base_all)
    return (out_user, out_item)


# speedup vs baseline: 7.5579x; 1.4410x over previous
import functools

import jax
import jax.numpy as jnp
from jax import lax
from jax.experimental import pallas as pl
from jax.experimental.pallas import tpu as pltpu
from jax.experimental.pallas import tpu_sc as plsc

N = 10000
D = 128
E = 320000
NPAD = 10240
NC = 2
NS = 16
B = 128
G = 8
NGRP = 20
NPAIR = NGRP // 2
CH = G * NGRP
EPT = CH * B
E_PAD = NS * EPT
RPT = 624
TAIL = N - NS * RPT


def _tc_transform(x_user_p, x_item_p, W_msg_u2i, W_self_u2i, W_msg_i2u, W_self_i2u):
    BLK = 256

    def body(xu_ref, xi_ref, wm0_ref, ws0_ref, wm1_ref, ws1_ref, y_ref, b_ref):
        xu = xu_ref[...]
        xi = xi_ref[...]
        y_ref[0] = jnp.dot(xu, wm0_ref[...], preferred_element_type=jnp.float32)
        y_ref[1] = jnp.dot(xi, wm1_ref[...], preferred_element_type=jnp.float32)
        b_ref[0] = jnp.dot(xi, ws0_ref[...], preferred_element_type=jnp.float32)
        b_ref[1] = jnp.dot(xu, ws1_ref[...], preferred_element_type=jnp.float32)

    grid = (NPAD // BLK,)
    w_spec = pl.BlockSpec((D, D), lambda i: (0, 0))
    return pl.pallas_call(
        body,
        grid=grid,
        in_specs=[
            pl.BlockSpec((BLK, D), lambda i: (i, 0)),
            pl.BlockSpec((BLK, D), lambda i: (i, 0)),
            w_spec, w_spec, w_spec, w_spec,
        ],
        out_specs=[
            pl.BlockSpec((NC, BLK, D), lambda i: (0, i, 0)),
            pl.BlockSpec((NC, BLK, D), lambda i: (0, i, 0)),
        ],
        out_shape=[
            jax.ShapeDtypeStruct((NC, NPAD, D), jnp.float32),
            jax.ShapeDtypeStruct((NC, NPAD, D), jnp.float32),
        ],
    )(x_user_p, x_item_p, W_msg_u2i, W_self_u2i, W_msg_i2u, W_self_i2u)


def _sc_conv(y_flat, e_src, e_dst, base_all):
    mesh = plsc.VectorSubcoreMesh(core_axis_name="c", subcore_axis_name="s")

    @functools.partial(
        pl.kernel,
        out_type=(
            jax.ShapeDtypeStruct((N, D), jnp.float32),
            jax.ShapeDtypeStruct((N, D), jnp.float32),
        ),
        mesh=mesh,
        scratch_types=[
            pltpu.VMEM((G, B), jnp.int32),
            pltpu.VMEM((G, B), jnp.int32),
            pltpu.VMEM((G, B), jnp.int32),
            pltpu.VMEM((G, B), jnp.int32),
            pltpu.VMEM((B, D), jnp.float32),
            pltpu.VMEM((B, D), jnp.float32),
            pltpu.SemaphoreType.DMA,
            pltpu.SemaphoreType.DMA,
            pltpu.SemaphoreType.DMA,
            pltpu.SemaphoreType.DMA,
            pltpu.VMEM_SHARED((N, D), jnp.float32),
        ],
    )
    def k(y_ref, src_ref, dst_ref, base_ref, out_user, out_item,
          sidx0, sidx1, didx0, didx1, r0, r1,
          sg0, sg1, si0, si1, acc):
        cid = lax.axis_index("c")
        sid = lax.axis_index("s")
        row0 = pl.multiple_of(sid * RPT, 8)
        pltpu.sync_copy(base_ref.at[cid, pl.ds(row0, RPT)], acc.at[pl.ds(row0, RPT)])

        @pl.when(sid == NS - 1)
        def _():
            pltpu.sync_copy(base_ref.at[cid, pl.ds(NS * RPT, TAIL)],
                            acc.at[pl.ds(NS * RPT, TAIL)])

        plsc.subcore_barrier()

        RW = (r0, r1)
        SG = (sg0, sg1)
        SIDX = (sidx0, sidx1)
        DIDX = (didx0, didx1)
        SI = (si0, si1)

        def refill(slot, h):
            h0 = pl.multiple_of(h * G, 8)
            pltpu.async_copy(src_ref.at[cid, sid, pl.ds(h0, G)], SIDX[slot], SI[slot])
            pltpu.async_copy(dst_ref.at[cid, sid, pl.ds(h0, G)], DIDX[slot], SI[slot])

        def wait_refill(slot):
            pltpu.make_async_copy(src_ref.at[cid, sid, pl.ds(0, G)],
                                  SIDX[slot], SI[slot]).wait()
            pltpu.make_async_copy(dst_ref.at[cid, sid, pl.ds(0, G)],
                                  DIDX[slot], SI[slot]).wait()

        def gather(slot, p, b):
            pltpu.async_copy(y_ref.at[SIDX[slot].at[p]], RW[b], SG[b])

        def wait_gather(slot, p, b):
            pltpu.make_async_copy(y_ref.at[SIDX[slot].at[p]], RW[b], SG[b]).wait()

        def chunk(gip, p, i, pred, first_pair):
            kk = 8 * gip + p
            b = kk % 2
            slot = gip
            if p == 2:
                if gip == 0:
                    refill(1, 2 * i + 1)
                elif first_pair:
                    refill(0, 2 * i + 2)
                else:
                    @pl.when(pred)
                    def _():
                        refill(0, 2 * i + 2)
            if p == 6:
                if gip == 0 or first_pair:
                    wait_refill(1 - slot)
                else:
                    @pl.when(pred)
                    def _():
                        wait_refill(1 - slot)
            wait_gather(slot, p, b)
            pltpu.sync_copy(RW[b], acc.at[DIDX[slot].at[p]], add=True)
            if p < 6:
                gather(slot, p + 2, b)
            elif gip == 0 or first_pair:
                gather(1 - slot, p - 6, b)
            else:
                @pl.when(pred)
                def _():
                    gather(1 - slot, p - 6, b)

        refill(0, 0)
        wait_refill(0)
        gather(0, 0, 0)
        gather(0, 1, 1)
        for gip in range(2):
            for p in range(G):
                chunk(gip, p, 0, None, True)

        def pair(i, carry):
            pred = i < NPAIR - 1
            for gip in range(2):
                for p in range(G):
                    chunk(gip, p, i, pred, False)
            return carry

        lax.fori_loop(1, NPAIR, pair, 0)
        plsc.subcore_barrier()

        @pl.when(cid == 0)
        def _():
            pltpu.sync_copy(acc.at[pl.ds(row0, RPT)], out_item.at[pl.ds(row0, RPT)])

            @pl.when(sid == NS - 1)
            def _():
                pltpu.sync_copy(acc.at[pl.ds(NS * RPT, TAIL)],
                                out_item.at[pl.ds(NS * RPT, TAIL)])

        @pl.when(cid == 1)
        def _():
            pltpu.sync_copy(acc.at[pl.ds(row0, RPT)], out_user.at[pl.ds(row0, RPT)])

            @pl.when(sid == NS - 1)
            def _():
                pltpu.sync_copy(acc.at[pl.ds(NS * RPT, TAIL)],
                                out_user.at[pl.ds(NS * RPT, TAIL)])

    return k(y_flat, e_src, e_dst, base_all)


def _prep_edges(edge_index_u2i, edge_index_i2u):
    src0 = edge_index_u2i[0].astype(jnp.int32)
    dst0 = edge_index_u2i[1].astype(jnp.int32)
    src1 = edge_index_i2u[0].astype(jnp.int32) + NPAD
    dst1 = edge_index_i2u[1].astype(jnp.int32)
    npad = E_PAD - E
    pad0 = jnp.full((npad,), N, jnp.int32)
    pad1 = jnp.full((npad,), NPAD + N, jnp.int32)
    padd = jnp.zeros((npad,), jnp.int32)
    e_src = jnp.stack([jnp.concatenate([src0, pad0]),
                       jnp.concatenate([src1, pad1])]).reshape(NC, NS, CH, B)
    e_dst = jnp.stack([jnp.concatenate([dst0, padd]),
                       jnp.concatenate([dst1, padd])]).reshape(NC, NS, CH, B)
    return e_src, e_dst


def kernel(x_user, x_item, edge_index_u2i, edge_index_i2u,
           W_msg_u2i, W_self_u2i, W_msg_i2u, W_self_i2u):
    x_user_p = jnp.pad(x_user, ((0, NPAD - N), (0, 0)))
    x_item_p = jnp.pad(x_item, ((0, NPAD - N), (0, 0)))
    e_src, e_dst = _prep_edges(edge_index_u2i, edge_index_i2u)
    y_all, base_all = _tc_transform(x_user_p, x_item_p,
                                    W_msg_u2i, W_self_u2i, W_msg_i2u, W_self_i2u)
    y_flat = y_all.reshape(NC * NPAD, D)
    out_user, out_item = _sc_conv(y_flat, e_src, e_dst, base_all)
    return (out_user, out_item)
